# baseline (speedup 1.0000x reference)
import jax
import jax.numpy as jnp
from jax.experimental import pallas as pl

LEVELS = 4
K = 1024
D = 32
N = 4096
BN = 256
CK = 128


def _rvq_kernel(x_ref, cb_ref, quant_ref, c0_ref, c1_ref, c2_ref, c3_ref):
    code_refs = (c0_ref, c1_ref, c2_ref, c3_ref)
    r = x_ref[...]
    quantized = jnp.zeros_like(r)
    iota_ck = jax.lax.broadcasted_iota(jnp.int32, (BN, CK), 1)
    for l in range(LEVELS):
        def kstep(k, carry):
            bmin, bidx, brow = carry
            cbc = cb_ref[l, pl.ds(k * CK, CK), :]
            s = jax.lax.dot_general(
                r, cbc, (((1,), (1,)), ((), ())),
                preferred_element_type=jnp.float32,
                precision=jax.lax.Precision.HIGHEST,
            )
            cbn = jnp.sum(cbc * cbc, axis=1)
            d = cbn[None, :] - 2.0 * s
            cmin = jnp.min(d, axis=1)
            cidx = jnp.min(
                jnp.where(d == cmin[:, None], iota_ck, CK), axis=1
            )
            onehot = (iota_ck == cidx[:, None]).astype(jnp.float32)
            crow = jax.lax.dot_general(
                onehot, cbc, (((1,), (0,)), ((), ())),
                preferred_element_type=jnp.float32,
                precision=jax.lax.Precision.HIGHEST,
            )
            upd = cmin < bmin
            bmin = jnp.where(upd, cmin, bmin)
            bidx = jnp.where(upd, cidx + k * CK, bidx)
            brow = jnp.where(upd[:, None], crow, brow)
            return bmin, bidx, brow

        init = (
            jnp.full((BN,), jnp.inf, jnp.float32),
            jnp.zeros((BN,), jnp.int32),
            jnp.zeros((BN, D), jnp.float32),
        )
        _, idx, quant = jax.lax.fori_loop(0, K // CK, kstep, init)
        code_refs[l][0, 0, :] = idx
        quantized = quantized + quant
        r = r - quant
    quant_ref[...] = quantized


@jax.jit
def kernel(x, codebooks):
    grid = (N // BN,)
    code_shape = jax.ShapeDtypeStruct((N // BN, 1, BN), jnp.int32)
    code_spec = pl.BlockSpec((1, 1, BN), lambda i: (i, 0, 0))
    quant, c0, c1, c2, c3 = pl.pallas_call(
        _rvq_kernel,
        grid=grid,
        in_specs=[
            pl.BlockSpec((BN, D), lambda i: (i, 0)),
            pl.BlockSpec((LEVELS, K, D), lambda i: (0, 0, 0)),
        ],
        out_specs=[
            pl.BlockSpec((BN, D), lambda i: (i, 0)),
            code_spec, code_spec, code_spec, code_spec,
        ],
        out_shape=[
            jax.ShapeDtypeStruct((N, D), jnp.float32),
            code_shape, code_shape, code_shape, code_shape,
        ],
    )(x, codebooks)
    codes = jnp.stack(
        [c.reshape(N) for c in (c0, c1, c2, c3)], axis=1
    )
    return quant, codes
